# baseline (speedup 1.0000x reference)
import jax
import jax.numpy as jnp
from jax import lax
from jax.experimental import pallas as pl
from jax.experimental.pallas import tpu as pltpu
from jax.experimental.pallas import tpu_sc as plsc

V = 1000000
D = 64
B = 16384

NC = 2
NS = 16
L = 16
NW = NC * NS
BPW = B // NW
G = BPW // L
DCH = D // L


def _glove_body(wi_hbm, wj_hbm, bi_hbm, bj_hbm, idx_hbm,
                out_hbm, idx_v, rows_i, rows_j, bvi, bvj,
                out_v, sem0, sem1):
    wid = lax.axis_index("s") * NC + lax.axis_index("c")
    base = wid * BPW

    pltpu.sync_copy(idx_hbm.at[wid], idx_v)

    def copies_for(g, buf):
        sem = sem0 if buf == 0 else sem1
        iv_i = idx_v[0, pl.ds(g * L, L)]
        iv_j = idx_v[1, pl.ds(g * L, L)]
        cps = []
        for r in range(L):
            w_i = iv_i[r]
            w_j = iv_j[r]
            cps.append(pltpu.make_async_copy(
                wi_hbm.at[w_i], rows_i.at[buf, r], sem))
            cps.append(pltpu.make_async_copy(
                wj_hbm.at[w_j], rows_j.at[buf, r], sem))
            cps.append(pltpu.make_async_copy(
                bi_hbm.at[w_i], bvi.at[buf, pl.ds(r, 1)], sem))
            cps.append(pltpu.make_async_copy(
                bj_hbm.at[w_j], bvj.at[buf, pl.ds(r, 1)], sem))
        return cps

    def fire(g, buf):
        for cp in copies_for(g, buf):
            cp.start()

    def drain(g, buf):
        for cp in copies_for(g, buf):
            cp.wait()

    lane = lax.iota(jnp.int32, L)
    dnums = lax.GatherDimensionNumbers(
        offset_dims=(), collapsed_slice_dims=(0,), start_index_map=(0,))

    def vperm(v, idx):
        return lax.gather(v, idx[:, None], dnums, slice_sizes=(1,),
                          mode=lax.GatherScatterMode.PROMISE_IN_BOUNDS)

    fire(0, 0)
    fire(1, 1)

    def group(g, carry):
        buf = lax.rem(g, 2)

        def with_buf(buf):
            drain(g, buf)

            @pl.when(g < G - 2)
            def _():
                fire(g + 2, buf)

            out16 = bvi[buf, pl.ds(0, L)] + bvj[buf, pl.ds(0, L)]
            for r in range(L):
                acc = None
                for cch in range(DCH):
                    sl = pl.ds(cch * L, L)
                    a = rows_i[buf, r, sl]
                    b = rows_j[buf, r, sl]
                    acc = a * b if acc is None else acc + a * b
                for sh in (8, 4, 2, 1):
                    acc = acc + vperm(acc, lane ^ sh)
                out16 = jnp.where(lane == r, out16 + acc, out16)
            out_v[pl.ds(g * L, L)] = out16

        lax.cond(buf == 0, lambda: with_buf(0), lambda: with_buf(1))
        return carry

    lax.fori_loop(0, G, group, 0)

    pltpu.sync_copy(out_v, out_hbm.at[pl.ds(base, BPW)])


@jax.jit
def _glove(idx2, wi, wj, bi, bj):
    mesh = plsc.VectorSubcoreMesh(core_axis_name="c", subcore_axis_name="s")
    run = pl.kernel(
        _glove_body,
        out_type=jax.ShapeDtypeStruct((B,), jnp.float32),
        mesh=mesh,
        scratch_types=[
            pltpu.VMEM((2, BPW), jnp.int32),
            pltpu.VMEM((2, L, D), jnp.float32),
            pltpu.VMEM((2, L, D), jnp.float32),
            pltpu.VMEM((2, L), jnp.float32),
            pltpu.VMEM((2, L), jnp.float32),
            pltpu.VMEM((BPW,), jnp.float32),
            pltpu.SemaphoreType.DMA,
            pltpu.SemaphoreType.DMA,
        ],
    )
    return run(wi, wj, bi, bj, idx2)


def kernel(word_i, word_j, wi, wj, bi, bj):
    wi32 = word_i.astype(jnp.int32)
    wj32 = word_j.astype(jnp.int32)
    idx2 = jnp.stack([wi32, wj32]).reshape(2, NW, BPW)
    idx2 = idx2.transpose(1, 0, 2)
    return _glove(idx2, wi, wj, bi, bj)
